# Initial kernel scaffold; baseline (speedup 1.0000x reference)
import jax
import jax.numpy as jnp
from jax.experimental import pallas as pl


def kernel(x, wave_w, wave_b, query_w, query_b, key_weight, out_w, se1_w, se1_b, se2_w, se2_b):
    raise NotImplementedError("write your pallas kernel here")


# speedup vs baseline: 7.6706x; 7.6706x over previous
import functools
import jax
import jax.numpy as jnp
from jax import lax
from jax.experimental import pallas as pl
from jax.experimental.pallas import tpu as pltpu
from jax.experimental.pallas import tpu_sc as plsc

C = 1024
H = 16
D = C // H
POS_DIM = 16
HALF_S = 16
S = 2 * HALF_S + 1
MAX_FREQ = 16.0
MIN_FREQ = 1.0
SCALE = POS_DIM ** -0.5

L = 2048
T1 = 256
T3 = 256
NC = 2
NS = 16
NW = NC * NS
TPW = L // NW
NBUF = 2


def _silu(v):
    return v * jax.nn.sigmoid(v)


def _stage1_body(x_ref, wwT_ref, wb_ref, qwT_ref, qb_ref, kwt_ref, seg_ref,
                 attn_ref, idx_ref):
    i = pl.program_id(0)
    xb = x_ref[...]
    wave = _silu(jnp.dot(xb, wwT_ref[...],
                         preferred_element_type=jnp.float32) + wb_ref[...])
    freq_avg = (jnp.mean(jax.nn.sigmoid(wave[:, 0:16]), axis=1, keepdims=True)
                * (MAX_FREQ - MIN_FREQ) + MIN_FREQ)
    phase_avg = jnp.mean(jnp.tanh(wave[:, 16:32]), axis=1,
                         keepdims=True) * MAX_FREQ
    decay_avg = (jnp.mean(jax.nn.sigmoid(wave[:, 32:48]), axis=1,
                          keepdims=True) * 9.5 + 0.5)

    q = _silu(jnp.dot(xb, qwT_ref[...],
                      preferred_element_type=jnp.float32) + qb_ref[...])
    qk = jnp.sum((q * kwt_ref[...]).reshape(T1, H, POS_DIM), axis=2)

    centers = (jnp.float32(i * T1)
               + lax.broadcasted_iota(jnp.int32, (T1, S), 0).astype(jnp.float32))
    stride = (lax.broadcasted_iota(jnp.int32, (T1, S), 1).astype(jnp.float32)
              - HALF_S)
    pos = centers + stride * freq_avg + phase_avg
    valid = (pos >= 0.0) & (pos < L)
    idx_ref[...] = jnp.clip(pos.astype(jnp.int32), 0, L - 1)

    rel = jnp.abs(stride) * freq_avg
    BIG = jnp.float32(1e30)
    relmax = jnp.max(jnp.where(valid, rel, -BIG), axis=1, keepdims=True)
    relmin = jnp.min(jnp.where(valid, rel, BIG), axis=1, keepdims=True)
    m = jnp.where(qk >= 0, qk * relmax, qk * relmin) * SCALE

    FW = S * H
    col = lax.broadcasted_iota(jnp.int32, (T1, FW), 1)
    sf = (col // H).astype(jnp.float32) - HALF_S
    relf = jnp.abs(sf) * freq_avg
    posf = centers[:, 0:1] + sf * freq_avg + phase_avg
    validf = ((posf >= 0.0) & (posf < L)).astype(jnp.float32)
    qk_f = jnp.concatenate([qk] * S, axis=1)
    m_f = jnp.concatenate([m] * S, axis=1)
    e = jnp.exp(jnp.minimum((qk_f * relf) * SCALE - m_f, 0.0)) * validf
    envf = jnp.exp(-relf / jnp.clip(decay_avg, 0.1, None))
    a = e * envf
    seg = seg_ref[...]
    ssum = jnp.dot(e, seg, preferred_element_type=jnp.float32)
    asum = jnp.dot(a, seg, preferred_element_type=jnp.float32)
    norm = asum + 1e-8 * ssum
    norm_f = jnp.concatenate([norm] * S, axis=1)
    attn_ref[...] = a / norm_f


def _stage1(xT, wwT, wb, qwT, qb, kwt, seg):
    return pl.pallas_call(
        _stage1_body,
        grid=(L // T1,),
        in_specs=[
            pl.BlockSpec((T1, C), lambda i: (i, 0)),
            pl.BlockSpec((C, 3 * H), lambda i: (0, 0)),
            pl.BlockSpec((1, 3 * H), lambda i: (0, 0)),
            pl.BlockSpec((C, H * POS_DIM), lambda i: (0, 0)),
            pl.BlockSpec((1, H * POS_DIM), lambda i: (0, 0)),
            pl.BlockSpec((1, H * POS_DIM), lambda i: (0, 0)),
            pl.BlockSpec((S * H, H), lambda i: (0, 0)),
        ],
        out_specs=[
            pl.BlockSpec((T1, S * H), lambda i: (i, 0)),
            pl.BlockSpec((T1, S), lambda i: (i, 0)),
        ],
        out_shape=[
            jax.ShapeDtypeStruct((L, S * H), jnp.float32),
            jax.ShapeDtypeStruct((L, S), jnp.int32),
        ],
    )(xT, wwT, wb, qwT, qb, kwt, seg)


def _sc_body(xT_hbm, idx_hbm, attn_hbm, out_hbm,
             idx_v, attn_v, rows_v, out_v, gsem, asem):
    wid = lax.axis_index("s") * NC + lax.axis_index("c")
    base = wid * TPW
    pltpu.sync_copy(idx_hbm.at[pl.ds(base, TPW)], idx_v)

    for b in range(NBUF):
        pltpu.async_copy(xT_hbm.at[idx_v.at[b]], rows_v.at[b], gsem.at[b])
        pltpu.async_copy(attn_hbm.at[base + b], attn_v.at[b], asem.at[b])

    def outer(t0, carry):
        for b in range(NBUF):
            t = t0 + b
            pltpu.make_async_copy(xT_hbm.at[idx_v.at[t]], rows_v.at[b],
                                  gsem.at[b]).wait()
            pltpu.make_async_copy(attn_hbm.at[base + t], attn_v.at[b],
                                  asem.at[b]).wait()

            avecs = [attn_v[b, pl.ds(s * H, H)] for s in range(S)]

            def jbody(j, c):
                off = pl.multiple_of(j * 16, 16)
                acc = avecs[0] * rows_v[b, 0, pl.ds(off, 16)]
                for s in range(1, S):
                    acc = acc + avecs[s] * rows_v[b, s, pl.ds(off, 16)]
                out_v[pl.ds(off, 16)] = acc
                return c

            lax.fori_loop(0, C // 16, jbody, 0, unroll=2)
            pltpu.sync_copy(out_v, out_hbm.at[base + t])

            @pl.when(t + NBUF < TPW)
            def _():
                pltpu.async_copy(xT_hbm.at[idx_v.at[t + NBUF]], rows_v.at[b],
                                 gsem.at[b])
                pltpu.async_copy(attn_hbm.at[base + t + NBUF], attn_v.at[b],
                                 asem.at[b])
        return carry

    lax.fori_loop(0, TPW // NBUF, lambda i, c: outer(i * NBUF, c), 0)


def _stage2(xT, sidx, attn):
    fn = pl.kernel(
        _sc_body,
        out_type=jax.ShapeDtypeStruct((L, C), jnp.float32),
        mesh=plsc.VectorSubcoreMesh(core_axis_name="c", subcore_axis_name="s"),
        scratch_types=[
            pltpu.VMEM((TPW, S), jnp.int32),
            pltpu.VMEM((NBUF, S * H), jnp.float32),
            pltpu.VMEM((NBUF, S, C), jnp.float32),
            pltpu.VMEM((C,), jnp.float32),
            pltpu.SemaphoreType.DMA((NBUF,)),
            pltpu.SemaphoreType.DMA((NBUF,)),
        ],
    )
    return fn(xT, sidx, attn)


def _stage3_body(g_ref, w1T_ref, b1_ref, w2T_ref, b2_ref, woT_ref, out_ref):
    g = g_ref[...]
    h1 = _silu(jnp.dot(g, w1T_ref[...],
                       preferred_element_type=jnp.float32) + b1_ref[...])
    se = jax.nn.sigmoid(jnp.dot(h1, w2T_ref[...],
                                preferred_element_type=jnp.float32)
                        + b2_ref[...])
    o = jnp.dot(g * se, woT_ref[...], preferred_element_type=jnp.float32)
    out_ref[...] = _silu(o)


def _stage3(agg, w1T, b1, w2T, b2, woT):
    return pl.pallas_call(
        _stage3_body,
        grid=(L // T3,),
        in_specs=[
            pl.BlockSpec((T3, C), lambda i: (i, 0)),
            pl.BlockSpec((C, C // 4), lambda i: (0, 0)),
            pl.BlockSpec((1, C // 4), lambda i: (0, 0)),
            pl.BlockSpec((C // 4, C), lambda i: (0, 0)),
            pl.BlockSpec((1, C), lambda i: (0, 0)),
            pl.BlockSpec((C, C), lambda i: (0, 0)),
        ],
        out_specs=pl.BlockSpec((T3, C), lambda i: (i, 0)),
        out_shape=jax.ShapeDtypeStruct((L, C), jnp.float32),
    )(agg, w1T, b1, w2T, b2, woT)


def _permute_cols(w):
    return w.reshape(-1, H, D).transpose(0, 2, 1).reshape(w.shape[0], C)


def kernel(x, wave_w, wave_b, query_w, query_b, key_weight, out_w,
           se1_w, se1_b, se2_w, se2_b):
    B = x.shape[0]
    xm = x.reshape(L, C)
    xT = xm.reshape(L, H, D).transpose(0, 2, 1).reshape(L, C)

    wwT = _permute_cols(wave_w).T
    qwT = _permute_cols(query_w).T
    kwt = jnp.tile(key_weight, H).reshape(1, H * POS_DIM)
    seg = (jnp.arange(S * H, dtype=jnp.int32)[:, None] % H
           == jnp.arange(H, dtype=jnp.int32)[None, :]).astype(jnp.float32)
    attn, sidx = _stage1(xT, wwT, wave_b.reshape(1, -1), qwT,
                         query_b.reshape(1, -1), kwt, seg)

    agg = _stage2(xT, sidx, attn)

    w1T = _permute_cols(se1_w).T
    w2T = se2_w.reshape(H, D, C // 4).transpose(1, 0, 2).reshape(C, C // 4).T
    b2 = se2_b.reshape(H, D).T.reshape(1, C)
    woT = _permute_cols(out_w).T
    out = _stage3(agg, w1T, se1_b.reshape(1, -1), w2T, b2, woT)
    return out.reshape(B, L, C)
